# baseline (speedup 1.0000x reference)
import functools

import jax
import jax.numpy as jnp
from jax import lax
from jax.experimental import pallas as pl
from jax.experimental.pallas import tpu as pltpu
from jax.experimental.pallas import tpu_sc as plsc

from jax._src import core as _jax_core
from jax._src.pallas.mosaic import sc_lowering as _scl
from jax.experimental.mosaic.dialects import tpu as _tpu_dialect

_lane_perm_p = _jax_core.Primitive("sc_lane_perm")


@_lane_perm_p.def_abstract_eval
def _lane_perm_abs(x, idx):
    return _jax_core.ShapedArray(x.shape, x.dtype)


@_scl.register_lowering_rule(_lane_perm_p)
def _lane_perm_lowering(ctx, x, idx):
    return _tpu_dialect.dynamic_gather(x, idx, dimensions=[0])


def _lane_perm(x, idx):
    return _lane_perm_p.bind(x, idx)

_VOCAB = 1000000
_DIM = 64
_B = 16384

_NC = 2
_NS = 16
_NW = _NC * _NS
_BPW = _B // _NW
_CHUNK = 128
_NCHUNK = _BPW // _CHUNK


def _sc_sim_kernel(cw_hbm, tw_hbm, win_hbm, wout_hbm, sim_hbm,
                   cidx_v, tidx_v, crows_v, trows_v, sim_v, sem):
    wid = lax.axis_index("s") * _NC + lax.axis_index("c")
    row0 = wid * _NCHUNK

    pltpu.sync_copy(cw_hbm.at[pl.ds(row0, _NCHUNK)], cidx_v)
    pltpu.sync_copy(tw_hbm.at[pl.ds(row0, _NCHUNK)], tidx_v)

    copies = []
    for j in range(_NCHUNK):
        copies.append(pltpu.async_copy(
            win_hbm.at[cidx_v.at[j]], crows_v.at[pl.ds(j * _CHUNK, _CHUNK)],
            sem))
        copies.append(pltpu.async_copy(
            wout_hbm.at[tidx_v.at[j]], trows_v.at[pl.ds(j * _CHUNK, _CHUNK)],
            sem))
    for c in copies:
        c.wait()

    lane = lax.iota(jnp.int32, 16)

    def body(b, _):
        vec = jnp.zeros((16,), jnp.float32)
        for r in range(16):
            i = b * 16 + r
            acc = crows_v[i, pl.ds(0, 16)] * trows_v[i, pl.ds(0, 16)]
            for k in range(1, _DIM // 16):
                acc += (crows_v[i, pl.ds(k * 16, 16)]
                        * trows_v[i, pl.ds(k * 16, 16)])
            for sh in (8, 4, 2, 1):
                acc = acc + _lane_perm(acc, lane ^ sh)
            vec = jnp.where(lane == r, acc, vec)
        sim_v[pl.ds(b * 16, 16)] = vec
        return 0

    lax.fori_loop(0, _BPW // 16, body, 0)

    pltpu.sync_copy(sim_v, sim_hbm.at[pl.ds(wid * _BPW, _BPW)])


@functools.partial(
    pl.kernel,
    out_type=jax.ShapeDtypeStruct((_B,), jnp.float32),
    mesh=plsc.VectorSubcoreMesh(core_axis_name="c", subcore_axis_name="s"),
    compiler_params=pltpu.CompilerParams(use_tc_tiling_on_sc=False),
    scratch_types=[
        pltpu.VMEM((_NCHUNK, _CHUNK), jnp.int32),
        pltpu.VMEM((_NCHUNK, _CHUNK), jnp.int32),
        pltpu.VMEM((_BPW, _DIM), jnp.float32),
        pltpu.VMEM((_BPW, _DIM), jnp.float32),
        pltpu.VMEM((_BPW,), jnp.float32),
        pltpu.SemaphoreType.DMA,
    ],
)
def _sc_sim(cw_hbm, tw_hbm, win_hbm, wout_hbm, sim_hbm,
            cidx_v, tidx_v, crows_v, trows_v, sim_v, sem):
    _sc_sim_kernel(cw_hbm, tw_hbm, win_hbm, wout_hbm, sim_hbm,
                   cidx_v, tidx_v, crows_v, trows_v, sim_v, sem)


def _tc_bce_kernel(sim_ref, y_ref, out_ref):
    s = sim_ref[...]
    y = y_ref[...]
    t = jnp.maximum(s, 0.0) - s * y + jnp.log1p(jnp.exp(-jnp.abs(s)))
    out_ref[...] = (jnp.sum(t) * (1.0 / _B))[None, None]


def kernel(center_words, target_words, label, W_in, W_out):
    cw = center_words.astype(jnp.int32).reshape(_NW * _NCHUNK, _CHUNK)
    tw = target_words.astype(jnp.int32).reshape(_NW * _NCHUNK, _CHUNK)
    sim = _sc_sim(cw, tw, W_in, W_out)
    loss = pl.pallas_call(
        _tc_bce_kernel,
        out_shape=jax.ShapeDtypeStruct((1, 1), jnp.float32),
    )(sim.reshape(128, 128), label.astype(jnp.float32).reshape(128, 128))
    return loss.reshape(())


# speedup vs baseline: 2.5457x; 2.5457x over previous
import functools

import jax
import jax.numpy as jnp
from jax import lax
from jax.experimental import pallas as pl
from jax.experimental.pallas import tpu as pltpu
from jax.experimental.pallas import tpu_sc as plsc

from jax._src import core as _jax_core
from jax._src.pallas.mosaic import sc_lowering as _scl
from jax.experimental.mosaic.dialects import tpu as _tpu_dialect

_lane_perm_p = _jax_core.Primitive("sc_lane_perm")


@_lane_perm_p.def_abstract_eval
def _lane_perm_abs(x, idx):
    return _jax_core.ShapedArray(x.shape, x.dtype)


@_scl.register_lowering_rule(_lane_perm_p)
def _lane_perm_lowering(ctx, x, idx):
    return _tpu_dialect.dynamic_gather(x, idx, dimensions=[0])


def _lane_perm(x, idx):
    return _lane_perm_p.bind(x, idx)


_VOCAB = 1000000
_DIM = 64
_B = 16384

_NC = 2
_NS = 16
_NW = _NC * _NS
_BPW = _B // _NW
_NGRP = _BPW // 16
_NBUF = 4


def _sc_sim_kernel(cw_hbm, tw_hbm, win_hbm, wout_hbm, sim_hbm,
                   cidx_v, tidx_v, cblk_v, tblk_v, sim_v, sim_sh,
                   csems, tsems):
    cid = lax.axis_index("c")
    sid = lax.axis_index("s")
    wid = cid * _NS + sid
    r = wid % 8

    slab = (wid // 8) * 8
    pltpu.sync_copy(cw_hbm.at[pl.ds(slab, 8)], cidx_v)
    pltpu.sync_copy(tw_hbm.at[pl.ds(slab, 8)], tidx_v)

    lane = lax.iota(jnp.int32, 16)

    def fire(v, tbl_hbm, blk_v, sems, slot):
        vb = pl.multiple_of((v >> 7) * 128, 128)
        return pltpu.async_copy(
            tbl_hbm.at[pl.ds(0, _DIM), pl.ds(vb, 128)],
            blk_v.at[slot], sems[slot])

    idxc0 = cidx_v[r, pl.ds(0, 16)]
    idxt0 = tidx_v[r, pl.ds(0, 16)]
    for j in range(_NBUF):
        fire(idxc0[j], win_hbm, cblk_v, csems, j)
        fire(idxt0[j], wout_hbm, tblk_v, tsems, j)

    def wait(tbl_hbm, blk_v, sems, slot):
        pltpu.make_async_copy(
            tbl_hbm.at[pl.ds(0, _DIM), pl.ds(0, 128)],
            blk_v.at[slot], sems[slot]).wait()

    def body(g, _):
        goff = g * 16
        noff = jnp.minimum(goff + 16, _BPW - 16)
        idxc = cidx_v[r, pl.ds(goff, 16)]
        idxt = tidx_v[r, pl.ds(goff, 16)]
        idxc_n = cidx_v[r, pl.ds(noff, 16)]
        idxt_n = tidx_v[r, pl.ds(noff, 16)]
        last = g == _NGRP - 1
        vec = jnp.zeros((16,), jnp.float32)
        for j in range(16):
            slot = j % _NBUF
            wait(win_hbm, cblk_v, csems, slot)
            wait(wout_hbm, tblk_v, tsems, slot)
            v_c = idxc[j]
            v_t = idxt[j]
            col_c, col_t = v_c & 127, v_t & 127
            b16c, b16t = col_c & 112, col_t & 112
            a, b = col_c & 15, col_t & 15
            rot = (b - a) & 15
            pidx = (lane + rot) & 15
            acc = jnp.zeros((16,), jnp.float32)
            for d in range(_DIM):
                cv = cblk_v[slot, d, pl.ds(b16c, 16)]
                tv = tblk_v[slot, d, pl.ds(b16t, 16)]
                acc = acc + cv * _lane_perm(tv, pidx)
            vec = jnp.where(lane == j, _lane_perm(acc, lane * 0 + a), vec)
            if j + _NBUF < 16:
                vn_c, vn_t = idxc[j + _NBUF], idxt[j + _NBUF]
            else:
                k = j + _NBUF - 16
                vn_c = jnp.where(last, idxc[15], idxc_n[k])
                vn_t = jnp.where(last, idxt[15], idxt_n[k])
            fire(vn_c, win_hbm, cblk_v, csems, slot)
            fire(vn_t, wout_hbm, tblk_v, tsems, slot)
        sim_v[pl.ds(goff, 16)] = vec
        return 0

    lax.fori_loop(0, _NGRP, body, 0)

    for j in range(_NBUF):
        wait(win_hbm, cblk_v, csems, j)
        wait(wout_hbm, tblk_v, tsems, j)

    pltpu.sync_copy(sim_v, sim_sh.at[pl.ds(sid * _BPW, _BPW)])
    plsc.subcore_barrier()

    @pl.when(sid == 0)
    def _():
        pltpu.sync_copy(sim_sh, sim_hbm.at[pl.ds(cid * (_B // _NC),
                                                 _B // _NC)])


@functools.partial(
    pl.kernel,
    out_type=jax.ShapeDtypeStruct((_B,), jnp.float32),
    mesh=plsc.VectorSubcoreMesh(core_axis_name="c", subcore_axis_name="s"),
    compiler_params=pltpu.CompilerParams(use_tc_tiling_on_sc=True),
    scratch_types=[
        pltpu.VMEM((8, _BPW), jnp.int32),
        pltpu.VMEM((8, _BPW), jnp.int32),
        pltpu.VMEM((_NBUF, _DIM, 128), jnp.float32),
        pltpu.VMEM((_NBUF, _DIM, 128), jnp.float32),
        pltpu.VMEM((_BPW,), jnp.float32),
        pltpu.VMEM_SHARED((_B // _NC,), jnp.float32),
        [pltpu.SemaphoreType.DMA] * _NBUF,
        [pltpu.SemaphoreType.DMA] * _NBUF,
    ],
)
def _sc_sim(cw_hbm, tw_hbm, win_hbm, wout_hbm, sim_hbm,
            cidx_v, tidx_v, cblk_v, tblk_v, sim_v, sim_sh, csems, tsems):
    _sc_sim_kernel(cw_hbm, tw_hbm, win_hbm, wout_hbm, sim_hbm,
                   cidx_v, tidx_v, cblk_v, tblk_v, sim_v, sim_sh,
                   csems, tsems)


def _tc_bce_kernel(sim_ref, y_ref, out_ref):
    s = sim_ref[...]
    y = y_ref[...]
    t = jnp.maximum(s, 0.0) - s * y + jnp.log1p(jnp.exp(-jnp.abs(s)))
    out_ref[...] = (jnp.sum(t) * (1.0 / _B))[None, None]


def kernel(center_words, target_words, label, W_in, W_out):
    cw = center_words.astype(jnp.int32).reshape(_NW, _BPW)
    tw = target_words.astype(jnp.int32).reshape(_NW, _BPW)
    sim = _sc_sim(cw, tw, W_in.T, W_out.T)
    loss = pl.pallas_call(
        _tc_bce_kernel,
        out_shape=jax.ShapeDtypeStruct((1, 1), jnp.float32),
    )(sim.reshape(128, 128), label.astype(jnp.float32).reshape(128, 128))
    return loss.reshape(())


# speedup vs baseline: 2.6454x; 1.0391x over previous
import functools

import jax
import jax.numpy as jnp
from jax import lax
from jax.experimental import pallas as pl
from jax.experimental.pallas import tpu as pltpu
from jax.experimental.pallas import tpu_sc as plsc

from jax._src import core as _jax_core
from jax._src.pallas.mosaic import sc_lowering as _scl
from jax.experimental.mosaic.dialects import tpu as _tpu_dialect

_lane_perm_p = _jax_core.Primitive("sc_lane_perm")


@_lane_perm_p.def_abstract_eval
def _lane_perm_abs(x, idx):
    return _jax_core.ShapedArray(x.shape, x.dtype)


@_scl.register_lowering_rule(_lane_perm_p)
def _lane_perm_lowering(ctx, x, idx):
    return _tpu_dialect.dynamic_gather(x, idx, dimensions=[0])


def _lane_perm(x, idx):
    return _lane_perm_p.bind(x, idx)


_VOCAB = 1000000
_DIM = 64
_B = 16384

_NC = 2
_NS = 16
_NW = _NC * _NS
_BPW = _B // _NW
_NGRP = _BPW // 16
_NBUF = 4


def _sc_sim_kernel(cw_hbm, tw_hbm, win_hbm, wout_hbm, sim_hbm,
                   cidx_v, tidx_v, cblk_v, tblk_v, sim_v, sim_sh,
                   csems, tsems):
    cid = lax.axis_index("c")
    sid = lax.axis_index("s")
    wid = cid * _NS + sid
    r = wid % 8

    slab = (wid // 8) * 8
    pltpu.sync_copy(cw_hbm.at[pl.ds(slab, 8)], cidx_v)
    pltpu.sync_copy(tw_hbm.at[pl.ds(slab, 8)], tidx_v)

    lane = lax.iota(jnp.int32, 16)

    def fire(v, tbl_hbm, blk_v, sems, slot):
        vb = pl.multiple_of(((v & 0xFFFFF) >> 7) * 128, 128)
        return pltpu.async_copy(
            tbl_hbm.at[pl.ds(0, _DIM), pl.ds(vb, 128)],
            blk_v.at[slot], sems.at[slot])

    def fire_c(pv):
        @pl.when(((pv >> 20) & 1) == 1)
        def _():
            fire(pv, win_hbm, cblk_v, csems, (pv >> 21) & (_NBUF - 1))

    def wait(tbl_hbm, blk_v, sems, slot):
        pltpu.make_async_copy(
            tbl_hbm.at[pl.ds(0, _DIM), pl.ds(0, 128)],
            blk_v.at[slot], sems.at[slot]).wait()

    def wait_c(pv):
        @pl.when(((pv >> 20) & 1) == 1)
        def _():
            wait(win_hbm, cblk_v, csems, (pv >> 21) & (_NBUF - 1))

    idxc0 = cidx_v[r, pl.ds(0, 16)]
    idxt0 = tidx_v[r, pl.ds(0, 16)]
    for j in range(_NBUF):
        fire_c(idxc0[j])
        fire(idxt0[j], wout_hbm, tblk_v, tsems, j)

    def body(g, _):
        goff = g * 16
        noff = jnp.minimum(goff + 16, _BPW - 16)
        idxc = cidx_v[r, pl.ds(goff, 16)]
        idxt = tidx_v[r, pl.ds(goff, 16)]
        idxc_n = cidx_v[r, pl.ds(noff, 16)]
        idxt_n = tidx_v[r, pl.ds(noff, 16)]
        last = g == _NGRP - 1
        vec = jnp.zeros((16,), jnp.float32)
        for j in range(16):
            slot = j % _NBUF
            pv = idxc[j]
            cslot = (pv >> 21) & (_NBUF - 1)
            wait_c(pv)
            wait(wout_hbm, tblk_v, tsems, slot)
            v_c = pv & 0xFFFFF
            v_t = idxt[j]
            col_c, col_t = v_c & 127, v_t & 127
            b16c, b16t = col_c & 112, col_t & 112
            a, b = col_c & 15, col_t & 15
            rot = (b - a) & 15
            pidx = (lane + rot) & 15
            acc = jnp.zeros((16,), jnp.float32)
            for d in range(_DIM):
                cv = cblk_v[cslot, d, pl.ds(b16c, 16)]
                tv = tblk_v[slot, d, pl.ds(b16t, 16)]
                acc = acc + cv * _lane_perm(tv, pidx)
            vec = jnp.where(lane == j, _lane_perm(acc, lane * 0 + a), vec)
            if j + _NBUF < 16:
                vn_c, vn_t = idxc[j + _NBUF], idxt[j + _NBUF]
            else:
                k = j + _NBUF - 16
                vn_c = jnp.where(last, 0, idxc_n[k])
                vn_t = jnp.where(last, idxt[15], idxt_n[k])
            fire_c(vn_c)
            fire(vn_t, wout_hbm, tblk_v, tsems, slot)
        sim_v[pl.ds(goff, 16)] = vec
        return 0

    lax.fori_loop(0, _NGRP, body, 0)

    for j in range(_NBUF):
        wait(wout_hbm, tblk_v, tsems, j)

    pltpu.sync_copy(sim_v, sim_sh.at[pl.ds(sid * _BPW, _BPW)])
    plsc.subcore_barrier()

    @pl.when(sid == 0)
    def _():
        pltpu.sync_copy(sim_sh, sim_hbm.at[pl.ds(cid * (_B // _NC),
                                                 _B // _NC)])


@functools.partial(
    pl.kernel,
    out_type=jax.ShapeDtypeStruct((_B,), jnp.float32),
    mesh=plsc.VectorSubcoreMesh(core_axis_name="c", subcore_axis_name="s"),
    compiler_params=pltpu.CompilerParams(use_tc_tiling_on_sc=True),
    scratch_types=[
        pltpu.VMEM((8, _BPW), jnp.int32),
        pltpu.VMEM((8, _BPW), jnp.int32),
        pltpu.VMEM((_NBUF, _DIM, 128), jnp.float32),
        pltpu.VMEM((_NBUF, _DIM, 128), jnp.float32),
        pltpu.VMEM((_BPW,), jnp.float32),
        pltpu.VMEM_SHARED((_B // _NC,), jnp.float32),
        pltpu.SemaphoreType.DMA((_NBUF,)),
        pltpu.SemaphoreType.DMA((_NBUF,)),
    ],
)
def _sc_sim(cw_hbm, tw_hbm, win_hbm, wout_hbm, sim_hbm,
            cidx_v, tidx_v, cblk_v, tblk_v, sim_v, sim_sh, csems, tsems):
    _sc_sim_kernel(cw_hbm, tw_hbm, win_hbm, wout_hbm, sim_hbm,
                   cidx_v, tidx_v, cblk_v, tblk_v, sim_v, sim_sh,
                   csems, tsems)


def _tc_bce_kernel(sim_ref, y_ref, out_ref):
    s = sim_ref[...]
    y = y_ref[...]
    t = jnp.maximum(s, 0.0) - s * y + jnp.log1p(jnp.exp(-jnp.abs(s)))
    out_ref[...] = (jnp.sum(t) * (1.0 / _B))[None, None]


def kernel(center_words, target_words, label, W_in, W_out):
    cw32 = center_words.astype(jnp.int32)
    perm = jnp.argsort(cw32)
    sv = cw32[perm].reshape(_NW, _BPW)
    tw = target_words.astype(jnp.int32)[perm].reshape(_NW, _BPW)
    yp = label[perm].astype(jnp.float32)
    prevb = jnp.concatenate(
        [jnp.full((_NW, 1), -1, jnp.int32), sv[:, :-1] >> 7], axis=1)
    nb = ((sv >> 7) != prevb).astype(jnp.int32)
    uslot = (jnp.cumsum(nb, axis=1) - 1) & (_NBUF - 1)
    pc = sv | (nb << 20) | (uslot << 21)
    sim = _sc_sim(pc, tw, W_in.T, W_out.T)
    loss = pl.pallas_call(
        _tc_bce_kernel,
        out_shape=jax.ShapeDtypeStruct((1, 1), jnp.float32),
    )(sim.reshape(128, 128), yp.reshape(128, 128))
    return loss.reshape(())
